# Initial kernel scaffold; baseline (speedup 1.0000x reference)
import jax
import jax.numpy as jnp
from jax.experimental import pallas as pl


def kernel(p, features, target):
    raise NotImplementedError("write your pallas kernel here")


# speedup vs baseline: 4.1352x; 4.1352x over previous
import functools

import jax
import jax.numpy as jnp
from jax import lax
from jax.experimental import pallas as pl
from jax.experimental.pallas import tpu as pltpu
from jax.experimental.pallas import tpu_sc as plsc

N = 8192
D = 128
K = 32
TEMP = 0.07
EPS = 1e-7

ROWS_A = 128
PTS_C = 256
NB = K - 1


def _topk_body(pr_ref, sr_ref, pc_ref, sc_ref, idx_ref):
    xr = pr_ref[:, 0:1].astype(jnp.float32)
    yr = pr_ref[:, 1:2].astype(jnp.float32)
    zr = pr_ref[:, 2:3].astype(jnp.float32)
    sr = sr_ref[:, 0:1]
    xc = pc_ref[0:1, :].astype(jnp.float32)
    yc = pc_ref[1:2, :].astype(jnp.float32)
    zc = pc_ref[2:3, :].astype(jnp.float32)
    sc = sc_ref[0:1, :]
    dot = xr * xc + yr * yc + zr * zc
    d2 = (sr + sc) - 2.0 * dot
    ii = lax.broadcasted_iota(jnp.int32, (ROWS_A, N), 1)
    kk = lax.broadcasted_iota(jnp.int32, (ROWS_A, K), 1)
    big = jnp.int32(2 ** 30)
    inf = jnp.float32(jnp.inf)

    def step(k, carry):
        d2, acc = carry
        m = jnp.min(d2, axis=1, keepdims=True)
        cand = jnp.where(d2 <= m, ii, big)
        j = jnp.min(cand, axis=1, keepdims=True)
        acc = jnp.where(kk == k, j, acc)
        d2 = jnp.where(ii == j, inf, d2)
        return (d2, acc)

    _, acc = lax.fori_loop(
        0, K, step, (d2, jnp.zeros((ROWS_A, K), jnp.int32)))
    idx_ref[:, :] = acc


def _topk(pr, sr, pc, sc):
    return pl.pallas_call(
        _topk_body,
        grid=(N // ROWS_A,),
        in_specs=[
            pl.BlockSpec((ROWS_A, 4), lambda i: (i, 0)),
            pl.BlockSpec((ROWS_A, 1), lambda i: (i, 0)),
            pl.BlockSpec((16, N), lambda i: (0, 0)),
            pl.BlockSpec((8, N), lambda i: (0, 0)),
        ],
        out_specs=pl.BlockSpec((ROWS_A, K), lambda i: (i, 0)),
        out_shape=jax.ShapeDtypeStruct((N, K), jnp.int32),
    )(pr, sr, pc, sc)


_SC_CHUNK = 128


def _sc_gather(idx_flat, table, target):
    info = plsc.get_sparse_core_info()
    nc, ns = info.num_cores, info.num_subcores
    nw = nc * ns
    total = N * NB
    per_w = total // nw
    n_chunks = per_w // _SC_CHUNK
    mesh = plsc.VectorSubcoreMesh(core_axis_name="c", subcore_axis_name="s")

    @functools.partial(
        pl.kernel,
        mesh=mesh,
        out_type=[
            jax.ShapeDtypeStruct((total, D), jnp.float32),
            jax.ShapeDtypeStruct((total,), jnp.int32),
        ],
        scratch_types=[
            pltpu.VMEM((_SC_CHUNK,), jnp.int32),
            pltpu.VMEM((_SC_CHUNK, D), jnp.float32),
            pltpu.VMEM((_SC_CHUNK,), jnp.int32),
            pltpu.SemaphoreType.DMA,
            pltpu.SemaphoreType.DMA,
        ],
    )
    def gather_k(idx_hbm, table_hbm, tgt_hbm, out_hbm, ntg_hbm,
                 idx_v, rows_v, ntg_v, sem, sem2):
        wid = lax.axis_index("s") * nc + lax.axis_index("c")
        base = wid * per_w

        def body(c, carry):
            off = base + c * _SC_CHUNK
            pltpu.sync_copy(idx_hbm.at[pl.ds(off, _SC_CHUNK)], idx_v)
            pltpu.async_copy(table_hbm.at[idx_v], rows_v, sem)
            pltpu.async_copy(tgt_hbm.at[idx_v], ntg_v, sem2).wait()
            pltpu.sync_copy(ntg_v, ntg_hbm.at[pl.ds(off, _SC_CHUNK)])
            pltpu.make_async_copy(table_hbm.at[idx_v], rows_v, sem).wait()
            pltpu.sync_copy(rows_v, out_hbm.at[pl.ds(off, _SC_CHUNK)])
            return carry

        lax.fori_loop(0, n_chunks, body, 0)

    return gather_k(idx_flat, table, target)


def _combine_body(feat_ref, g_ref, ntg_ref, tgt_ref, ls_ref, cn_ref):
    i = pl.program_id(0)
    feat = feat_ref[:, :]
    g = g_ref[:, :, :]
    diff = feat[:, None, :] - g
    s = jnp.sum(diff * diff, axis=-1)
    pm = (ntg_ref[:, :] == tgt_ref[:, :]).astype(jnp.float32)
    cnt = jnp.sum(pm, axis=1, keepdims=True)
    pointmask = jnp.logical_and(cnt > 0.0, cnt < float(NB)).astype(jnp.float32)

    dist = jnp.sqrt(s) + EPS
    d = -dist
    d = d - jnp.max(d, axis=1, keepdims=True)
    e = jnp.exp(d / TEMP)
    pos = jnp.sum(e * pm, axis=1, keepdims=True)
    neg = jnp.sum(e, axis=1, keepdims=True)
    lp = -jnp.log(pos / neg + EPS)

    ls = jnp.sum(lp * pointmask, axis=(0, 1), keepdims=True)
    cn = jnp.sum(pointmask, axis=(0, 1), keepdims=True)

    @pl.when(i == 0)
    def _():
        ls_ref[:, :] = jnp.zeros((1, 1), jnp.float32)
        cn_ref[:, :] = jnp.zeros((1, 1), jnp.float32)

    ls_ref[:, :] += ls
    cn_ref[:, :] += cn


def _combine(features, g3, ntg, tgt):
    return pl.pallas_call(
        _combine_body,
        grid=(N // PTS_C,),
        in_specs=[
            pl.BlockSpec((PTS_C, D), lambda i: (i, 0)),
            pl.BlockSpec((PTS_C, NB, D), lambda i: (i, 0, 0)),
            pl.BlockSpec((PTS_C, NB), lambda i: (i, 0)),
            pl.BlockSpec((PTS_C, 1), lambda i: (i, 0)),
        ],
        out_specs=[
            pl.BlockSpec((1, 1), lambda i: (0, 0)),
            pl.BlockSpec((1, 1), lambda i: (0, 0)),
        ],
        out_shape=[
            jax.ShapeDtypeStruct((1, 1), jnp.float32),
            jax.ShapeDtypeStruct((1, 1), jnp.float32),
        ],
    )(features, g3, ntg, tgt)


def kernel(p, features, target):
    sq = jnp.sum(p * p, axis=1)
    pb = p.astype(jnp.bfloat16)
    pr = jnp.pad(pb, ((0, 0), (0, 1)))
    pc = jnp.zeros((16, N), jnp.bfloat16).at[0:3, :].set(pb.T)
    sr = sq[:, None]
    sc = jnp.zeros((8, N), jnp.float32).at[0, :].set(sq)
    idx32 = _topk(pr, sr, pc, sc)
    nidx = idx32[:, 1:].reshape(-1)

    gathered, ntg = _sc_gather(nidx, features, target)
    g3 = gathered.reshape(N, NB, D)
    ntg2 = ntg.reshape(N, NB)

    ls, cn = _combine(features, g3, ntg2, target[:, None])
    return (ls[0, 0] / jnp.maximum(cn[0, 0], 1.0))


# speedup vs baseline: 4.4175x; 1.0683x over previous
import functools

import jax
import jax.numpy as jnp
from jax import lax
from jax.experimental import pallas as pl
from jax.experimental.pallas import tpu as pltpu
from jax.experimental.pallas import tpu_sc as plsc

N = 8192
D = 128
K = 32
TEMP = 0.07
EPS = 1e-7

ROWS_A = 128
PTS_C = 256
NB = K - 1


def _topk_body(pr_ref, sr_ref, pc_ref, sc_ref, idx_ref):
    xr = pr_ref[:, 0:1].astype(jnp.float32)
    yr = pr_ref[:, 1:2].astype(jnp.float32)
    zr = pr_ref[:, 2:3].astype(jnp.float32)
    sr = sr_ref[:, 0:1]
    xc = pc_ref[0:1, :].astype(jnp.float32)
    yc = pc_ref[1:2, :].astype(jnp.float32)
    zc = pc_ref[2:3, :].astype(jnp.float32)
    sc = sc_ref[0:1, :]
    dot = xr * xc + yr * yc + zr * zc
    d2 = (sr + sc) - 2.0 * dot
    ii = lax.broadcasted_iota(jnp.int32, (ROWS_A, N), 1)
    kk = lax.broadcasted_iota(jnp.int32, (ROWS_A, K), 1)
    big = jnp.int32(2 ** 30)
    inf = jnp.float32(jnp.inf)

    def step(k, carry):
        d2, acc = carry
        j = jnp.argmin(d2, axis=1).astype(jnp.int32)[:, None]
        acc = jnp.where(kk == k, j, acc)
        d2 = jnp.where(ii == j, inf, d2)
        return (d2, acc)

    _, acc = lax.fori_loop(
        0, K, step, (d2, jnp.zeros((ROWS_A, K), jnp.int32)))
    idx_ref[:, :] = acc


def _topk(pr, sr, pc, sc):
    return pl.pallas_call(
        _topk_body,
        grid=(N // ROWS_A,),
        in_specs=[
            pl.BlockSpec((ROWS_A, 4), lambda i: (i, 0)),
            pl.BlockSpec((ROWS_A, 1), lambda i: (i, 0)),
            pl.BlockSpec((16, N), lambda i: (0, 0)),
            pl.BlockSpec((8, N), lambda i: (0, 0)),
        ],
        out_specs=pl.BlockSpec((ROWS_A, K), lambda i: (i, 0)),
        out_shape=jax.ShapeDtypeStruct((N, K), jnp.int32),
    )(pr, sr, pc, sc)


_SC_CHUNK = 128


def _sc_gather(idx_flat, table, target):
    info = plsc.get_sparse_core_info()
    nc, ns = info.num_cores, info.num_subcores
    nw = nc * ns
    total = N * NB
    per_w = total // nw
    n_chunks = per_w // _SC_CHUNK
    mesh = plsc.VectorSubcoreMesh(core_axis_name="c", subcore_axis_name="s")

    @functools.partial(
        pl.kernel,
        mesh=mesh,
        out_type=[
            jax.ShapeDtypeStruct((total, D), jnp.float32),
            jax.ShapeDtypeStruct((total,), jnp.int32),
        ],
        scratch_types=[
            pltpu.VMEM((_SC_CHUNK,), jnp.int32),
            pltpu.VMEM((_SC_CHUNK, D), jnp.float32),
            pltpu.VMEM((_SC_CHUNK,), jnp.int32),
            pltpu.SemaphoreType.DMA,
            pltpu.SemaphoreType.DMA,
        ],
    )
    def gather_k(idx_hbm, table_hbm, tgt_hbm, out_hbm, ntg_hbm,
                 idx_v, rows_v, ntg_v, sem, sem2):
        wid = lax.axis_index("s") * nc + lax.axis_index("c")
        base = wid * per_w

        def body(c, carry):
            off = base + c * _SC_CHUNK
            pltpu.sync_copy(idx_hbm.at[pl.ds(off, _SC_CHUNK)], idx_v)
            pltpu.async_copy(table_hbm.at[idx_v], rows_v, sem)
            pltpu.async_copy(tgt_hbm.at[idx_v], ntg_v, sem2).wait()
            pltpu.sync_copy(ntg_v, ntg_hbm.at[pl.ds(off, _SC_CHUNK)])
            pltpu.make_async_copy(table_hbm.at[idx_v], rows_v, sem).wait()
            pltpu.sync_copy(rows_v, out_hbm.at[pl.ds(off, _SC_CHUNK)])
            return carry

        lax.fori_loop(0, n_chunks, body, 0)

    return gather_k(idx_flat, table, target)


def _combine_body(feat_ref, g_ref, ntg_ref, tgt_ref, ls_ref, cn_ref):
    i = pl.program_id(0)
    feat = feat_ref[:, :]
    g = g_ref[:, :, :]
    diff = feat[:, None, :] - g
    s = jnp.sum(diff * diff, axis=-1)
    pm = (ntg_ref[:, :] == tgt_ref[:, :]).astype(jnp.float32)
    cnt = jnp.sum(pm, axis=1, keepdims=True)
    pointmask = jnp.logical_and(cnt > 0.0, cnt < float(NB)).astype(jnp.float32)

    dist = jnp.sqrt(s) + EPS
    d = -dist
    d = d - jnp.max(d, axis=1, keepdims=True)
    e = jnp.exp(d / TEMP)
    pos = jnp.sum(e * pm, axis=1, keepdims=True)
    neg = jnp.sum(e, axis=1, keepdims=True)
    lp = -jnp.log(pos / neg + EPS)

    ls = jnp.sum(lp * pointmask, axis=(0, 1), keepdims=True)
    cn = jnp.sum(pointmask, axis=(0, 1), keepdims=True)

    @pl.when(i == 0)
    def _():
        ls_ref[:, :] = jnp.zeros((1, 1), jnp.float32)
        cn_ref[:, :] = jnp.zeros((1, 1), jnp.float32)

    ls_ref[:, :] += ls
    cn_ref[:, :] += cn


def _combine(features, g3, ntg, tgt):
    return pl.pallas_call(
        _combine_body,
        grid=(N // PTS_C,),
        in_specs=[
            pl.BlockSpec((PTS_C, D), lambda i: (i, 0)),
            pl.BlockSpec((PTS_C, NB, D), lambda i: (i, 0, 0)),
            pl.BlockSpec((PTS_C, NB), lambda i: (i, 0)),
            pl.BlockSpec((PTS_C, 1), lambda i: (i, 0)),
        ],
        out_specs=[
            pl.BlockSpec((1, 1), lambda i: (0, 0)),
            pl.BlockSpec((1, 1), lambda i: (0, 0)),
        ],
        out_shape=[
            jax.ShapeDtypeStruct((1, 1), jnp.float32),
            jax.ShapeDtypeStruct((1, 1), jnp.float32),
        ],
    )(features, g3, ntg, tgt)


def kernel(p, features, target):
    sq = jnp.sum(p * p, axis=1)
    pb = p.astype(jnp.bfloat16)
    pr = jnp.pad(pb, ((0, 0), (0, 1)))
    pc = jnp.zeros((16, N), jnp.bfloat16).at[0:3, :].set(pb.T)
    sr = sq[:, None]
    sc = jnp.zeros((8, N), jnp.float32).at[0, :].set(sq)
    idx32 = _topk(pr, sr, pc, sc)
    nidx = idx32[:, 1:].reshape(-1)

    gathered, ntg = _sc_gather(nidx, features, target)
    g3 = gathered.reshape(N, NB, D)
    ntg2 = ntg.reshape(N, NB)

    ls, cn = _combine(features, g3, ntg2, target[:, None])
    return (ls[0, 0] / jnp.maximum(cn[0, 0], 1.0))


# speedup vs baseline: 4.4380x; 1.0046x over previous
import functools

import jax
import jax.numpy as jnp
from jax import lax
from jax.experimental import pallas as pl
from jax.experimental.pallas import tpu as pltpu
from jax.experimental.pallas import tpu_sc as plsc

N = 8192
D = 128
K = 32
TEMP = 0.07
EPS = 1e-7

ROWS_A = 256
PTS_C = 256
NB = K - 1


def _topk_body(pr_ref, sr_ref, pc_ref, sc_ref, idx_ref):
    xr = pr_ref[:, 0:1].astype(jnp.float32)
    yr = pr_ref[:, 1:2].astype(jnp.float32)
    zr = pr_ref[:, 2:3].astype(jnp.float32)
    sr = sr_ref[:, 0:1]
    xc = pc_ref[0:1, :].astype(jnp.float32)
    yc = pc_ref[1:2, :].astype(jnp.float32)
    zc = pc_ref[2:3, :].astype(jnp.float32)
    sc = sc_ref[0:1, :]
    dot = xr * xc + yr * yc + zr * zc
    d2 = (sr + sc) - 2.0 * dot
    ii = lax.broadcasted_iota(jnp.int32, (ROWS_A, N), 1)
    kk = lax.broadcasted_iota(jnp.int32, (ROWS_A, K), 1)
    big = jnp.int32(2 ** 30)
    inf = jnp.float32(jnp.inf)

    def step(k, carry):
        d2, acc = carry
        j = jnp.argmin(d2, axis=1).astype(jnp.int32)[:, None]
        acc = jnp.where(kk == k, j, acc)
        d2 = jnp.where(ii == j, inf, d2)
        return (d2, acc)

    _, acc = lax.fori_loop(
        0, K, step, (d2, jnp.zeros((ROWS_A, K), jnp.int32)))
    idx_ref[:, :] = acc


def _topk(pr, sr, pc, sc):
    return pl.pallas_call(
        _topk_body,
        grid=(N // ROWS_A,),
        in_specs=[
            pl.BlockSpec((ROWS_A, 4), lambda i: (i, 0)),
            pl.BlockSpec((ROWS_A, 1), lambda i: (i, 0)),
            pl.BlockSpec((16, N), lambda i: (0, 0)),
            pl.BlockSpec((8, N), lambda i: (0, 0)),
        ],
        out_specs=pl.BlockSpec((ROWS_A, K), lambda i: (i, 0)),
        out_shape=jax.ShapeDtypeStruct((N, K), jnp.int32),
    )(pr, sr, pc, sc)


_SC_CHUNK = 128


def _sc_gather(idx_flat, table, target):
    info = plsc.get_sparse_core_info()
    nc, ns = info.num_cores, info.num_subcores
    nw = nc * ns
    total = N * NB
    per_w = total // nw
    n_chunks = per_w // _SC_CHUNK
    mesh = plsc.VectorSubcoreMesh(core_axis_name="c", subcore_axis_name="s")

    @functools.partial(
        pl.kernel,
        mesh=mesh,
        out_type=[
            jax.ShapeDtypeStruct((total, D), jnp.float32),
            jax.ShapeDtypeStruct((total,), jnp.int32),
        ],
        scratch_types=[
            pltpu.VMEM((_SC_CHUNK,), jnp.int32),
            pltpu.VMEM((_SC_CHUNK, D), jnp.float32),
            pltpu.VMEM((_SC_CHUNK,), jnp.int32),
            pltpu.SemaphoreType.DMA,
            pltpu.SemaphoreType.DMA,
        ],
    )
    def gather_k(idx_hbm, table_hbm, tgt_hbm, out_hbm, ntg_hbm,
                 idx_v, rows_v, ntg_v, sem, sem2):
        wid = lax.axis_index("s") * nc + lax.axis_index("c")
        base = wid * per_w

        def body(c, carry):
            off = base + c * _SC_CHUNK
            pltpu.sync_copy(idx_hbm.at[pl.ds(off, _SC_CHUNK)], idx_v)
            pltpu.async_copy(table_hbm.at[idx_v], rows_v, sem)
            pltpu.async_copy(tgt_hbm.at[idx_v], ntg_v, sem2).wait()
            pltpu.sync_copy(ntg_v, ntg_hbm.at[pl.ds(off, _SC_CHUNK)])
            pltpu.make_async_copy(table_hbm.at[idx_v], rows_v, sem).wait()
            pltpu.sync_copy(rows_v, out_hbm.at[pl.ds(off, _SC_CHUNK)])
            return carry

        lax.fori_loop(0, n_chunks, body, 0)

    return gather_k(idx_flat, table, target)


def _combine_body(feat_ref, g_ref, ntg_ref, tgt_ref, ls_ref, cn_ref):
    i = pl.program_id(0)
    feat = feat_ref[:, :]
    g = g_ref[:, :, :]
    diff = feat[:, None, :] - g
    s = jnp.sum(diff * diff, axis=-1)
    pm = (ntg_ref[:, :] == tgt_ref[:, :]).astype(jnp.float32)
    cnt = jnp.sum(pm, axis=1, keepdims=True)
    pointmask = jnp.logical_and(cnt > 0.0, cnt < float(NB)).astype(jnp.float32)

    dist = jnp.sqrt(s) + EPS
    d = -dist
    d = d - jnp.max(d, axis=1, keepdims=True)
    e = jnp.exp(d / TEMP)
    pos = jnp.sum(e * pm, axis=1, keepdims=True)
    neg = jnp.sum(e, axis=1, keepdims=True)
    lp = -jnp.log(pos / neg + EPS)

    ls = jnp.sum(lp * pointmask, axis=(0, 1), keepdims=True)
    cn = jnp.sum(pointmask, axis=(0, 1), keepdims=True)

    @pl.when(i == 0)
    def _():
        ls_ref[:, :] = jnp.zeros((1, 1), jnp.float32)
        cn_ref[:, :] = jnp.zeros((1, 1), jnp.float32)

    ls_ref[:, :] += ls
    cn_ref[:, :] += cn


def _combine(features, g3, ntg, tgt):
    return pl.pallas_call(
        _combine_body,
        grid=(N // PTS_C,),
        in_specs=[
            pl.BlockSpec((PTS_C, D), lambda i: (i, 0)),
            pl.BlockSpec((PTS_C, NB, D), lambda i: (i, 0, 0)),
            pl.BlockSpec((PTS_C, NB), lambda i: (i, 0)),
            pl.BlockSpec((PTS_C, 1), lambda i: (i, 0)),
        ],
        out_specs=[
            pl.BlockSpec((1, 1), lambda i: (0, 0)),
            pl.BlockSpec((1, 1), lambda i: (0, 0)),
        ],
        out_shape=[
            jax.ShapeDtypeStruct((1, 1), jnp.float32),
            jax.ShapeDtypeStruct((1, 1), jnp.float32),
        ],
    )(features, g3, ntg, tgt)


def kernel(p, features, target):
    sq = jnp.sum(p * p, axis=1)
    pb = p.astype(jnp.bfloat16)
    pr = jnp.pad(pb, ((0, 0), (0, 1)))
    pc = jnp.zeros((16, N), jnp.bfloat16).at[0:3, :].set(pb.T)
    sr = sq[:, None]
    sc = jnp.zeros((8, N), jnp.float32).at[0, :].set(sq)
    idx32 = _topk(pr, sr, pc, sc)
    nidx = idx32[:, 1:].reshape(-1)

    gathered, ntg = _sc_gather(nidx, features, target)
    g3 = gathered.reshape(N, NB, D)
    ntg2 = ntg.reshape(N, NB)

    ls, cn = _combine(features, g3, ntg2, target[:, None])
    return (ls[0, 0] / jnp.maximum(cn[0, 0], 1.0))
